# baseline (speedup 1.0000x reference)
import functools

import numpy as np
import jax
import jax.numpy as jnp
from jax import lax
from jax.experimental import pallas as pl
from jax.experimental.pallas import tpu as pltpu
from jax.experimental.pallas import tpu_sc as plsc

B = 8
Q = 20000
C = 91
N = Q * C
NC, NS, L = 2, 16, 16
NW = NC * NS
SPB = NW // B
VPB = N // L
VPW = -(-VPB // SPB)
SPAN = VPW * L
CHUNK_V = 512
FULL_CHUNKS, TAIL_V = divmod(VPW, CHUNK_V)
NBINS = 1 << 16
CAP = 8192
NCAND = SPB * CAP
RROW = NCAND // 128
PAD_BITS = int(np.float32(-1e30).view(np.int32))
PAD_IDX = 1 << 30

_MESH = plsc.VectorSubcoreMesh(core_axis_name="c", subcore_axis_name="s",
                               num_cores=NC, num_subcores=NS)
_SC_PARAMS = pltpu.CompilerParams(needs_layout_passes=False)


def _worker():
    c = lax.axis_index("c")
    s = lax.axis_index("s")
    wid = c * NS + s
    b = wid // SPB
    w = wid % SPB
    dma_base = jnp.minimum(w * VPW, VPB - VPW) * L
    own_start = w * SPAN
    return b, w, dma_base, own_start


def _mono_key(bi):
    return bi ^ (jnp.right_shift(bi, 31) & jnp.int32(0x7FFFFFFF))


PRE_V = SPB * VPW - VPB
UNROLL = 4


def _hist_body(lg, hist_out, stage, hist_v):
    b, w, dma_base, own_start = _worker()
    zeros = jnp.zeros((L,), jnp.int32)
    ones = jnp.ones((L,), jnp.int32)

    def zbody(i, _):
        hist_v[pl.ds(i * L, L)] = zeros
        return 0
    lax.fori_loop(0, NBINS // L, zbody, 0)

    def vstep(j, mask=None):
        v = stage[pl.ds(j * L, L)]
        binv = jnp.right_shift(_mono_key(v), 16) + jnp.int32(32768)
        if mask is None:
            plsc.addupdate_scatter(hist_v, [binv], ones)
        else:
            plsc.addupdate_scatter(hist_v, [binv], ones, mask=mask)

    def process(chunk_v, nv, first=False):
        pltpu.sync_copy(lg.at[pl.ds(b * N + dma_base + chunk_v * L, nv * L)],
                        stage.at[pl.ds(0, nv * L)])
        lo = 0
        if first:
            for j in range(UNROLL):
                valid = jnp.broadcast_to(dma_base + j * L >= own_start, (L,))
                vstep(j, mask=valid)
            lo = UNROLL
        n4 = (nv - lo) // UNROLL

        def vbody(i, _):
            j0 = lo + i * UNROLL
            for u in range(UNROLL):
                vstep(j0 + u)
            return 0
        lax.fori_loop(0, n4, vbody, 0)
        for j in range(lo + n4 * UNROLL, nv):
            vstep(j)

    process(0, CHUNK_V, first=True)

    def outer(i, _):
        process(i * CHUNK_V, CHUNK_V)
        return 0
    lax.fori_loop(1, FULL_CHUNKS, outer, 0)
    process(FULL_CHUNKS * CHUNK_V, TAIL_V)

    wid = b * SPB + w
    pltpu.sync_copy(hist_v, hist_out.at[pl.ds(wid * NBINS, NBINS)])


_hist_call = functools.partial(
    pl.kernel,
    out_type=jax.ShapeDtypeStruct((NW * NBINS,), jnp.int32),
    mesh=_MESH,
    compiler_params=_SC_PARAMS,
    scratch_types=[pltpu.VMEM((CHUNK_V * L,), jnp.int32),
                   pltpu.VMEM((NBINS,), jnp.int32)],
)(_hist_body)


def _compact_body(lg, thr, cval_out, cidx_out, stage, cv, ci, thr_v):
    b, w, dma_base, own_start = _worker()
    iota = lax.iota(jnp.int32, L)
    pltpu.sync_copy(thr.at[pl.ds(b * L, L)], thr_v)
    tv = thr_v[...]

    padv = jnp.full((L,), PAD_BITS, jnp.int32)
    padi = jnp.full((L,), PAD_IDX, jnp.int32)

    def pbody(i, _):
        cv[pl.ds(i * L, L)] = padv
        ci[pl.ds(i * L, L)] = padi
        return 0
    lax.fori_loop(0, (CAP + L) // L, pbody, 0)

    def vstep(chunk_v, j, ptr, first=False):
        v = stage[pl.ds(j * L, L)]
        su = _mono_key(v)
        g0 = dma_base + (chunk_v + j) * L
        gidx = jnp.broadcast_to(g0, (L,)) + iota
        keep = su >= tv
        if first:
            keep = keep & jnp.broadcast_to(g0 >= own_start, (L,))
        plsc.store_compressed(cv.at[pl.ds(ptr, L)], v, mask=keep)
        plsc.store_compressed(ci.at[pl.ds(ptr, L)], gidx, mask=keep)
        cnt = jnp.sum(keep.astype(jnp.int32))
        return jnp.minimum(ptr + cnt, CAP)

    def process(chunk_v, nv, ptr, first=False):
        pltpu.sync_copy(lg.at[pl.ds(b * N + dma_base + chunk_v * L, nv * L)],
                        stage.at[pl.ds(0, nv * L)])
        lo = 0
        if first:
            for j in range(UNROLL):
                ptr = vstep(chunk_v, j, ptr, first=True)
            lo = UNROLL
        n4 = (nv - lo) // UNROLL

        def vbody(i, ptr):
            j0 = lo + i * UNROLL
            for u in range(UNROLL):
                ptr = vstep(chunk_v, j0 + u, ptr)
            return ptr
        ptr = lax.fori_loop(0, n4, vbody, ptr)
        for j in range(lo + n4 * UNROLL, nv):
            ptr = vstep(chunk_v, j, ptr)
        return ptr

    ptr = process(0, CHUNK_V, 0, first=True)

    def outer(i, ptr):
        return process(i * CHUNK_V, CHUNK_V, ptr)
    ptr = lax.fori_loop(1, FULL_CHUNKS, outer, ptr)
    process(FULL_CHUNKS * CHUNK_V, TAIL_V, ptr)

    pltpu.sync_copy(cv.at[pl.ds(0, CAP)],
                    cval_out.at[pl.ds(b * NCAND + w * CAP, CAP)])
    pltpu.sync_copy(ci.at[pl.ds(0, CAP)],
                    cidx_out.at[pl.ds(b * NCAND + w * CAP, CAP)])


_compact_call = functools.partial(
    pl.kernel,
    out_type=[jax.ShapeDtypeStruct((B * NCAND,), jnp.int32),
              jax.ShapeDtypeStruct((B * NCAND,), jnp.int32)],
    mesh=_MESH,
    compiler_params=_SC_PARAMS,
    scratch_types=[pltpu.VMEM((CHUNK_V * L,), jnp.int32),
                   pltpu.VMEM((CAP + L,), jnp.int32),
                   pltpu.VMEM((CAP + L,), jnp.int32),
                   pltpu.VMEM((L,), jnp.int32)],
)(_compact_body)


def _sort_body(s_ref, i_ref, os_ref, ol_ref, ob_ref):
    sv = s_ref[...]
    iv = i_ref[...]
    shape = (1, RROW, 128)
    lane = lax.broadcasted_iota(jnp.int32, shape, 2)
    pos = lax.broadcasted_iota(jnp.int32, shape, 1) * 128 + lane

    def flatroll(x, sh):
        ls = sh % 128
        m = (sh - ls) // 128
        y = x
        if ls:
            y = jnp.roll(y, ls, axis=2)
            y = jnp.where(lane >= ls, y, jnp.roll(y, 1, axis=1))
        if m:
            y = jnp.roll(y, m, axis=1)
        return y

    k = 2
    while k <= NCAND:
        j = k // 2
        while j >= 1:
            up = (pos & j) != 0
            desc = (pos & k) != 0
            sp = jnp.where(up, flatroll(sv, j), flatroll(sv, -j))
            ip = jnp.where(up, flatroll(iv, j), flatroll(iv, -j))
            prec = (sv > sp) | ((sv == sp) & (iv < ip))
            keep = prec ^ up ^ desc
            sv = jnp.where(keep, sv, sp)
            iv = jnp.where(keep, iv, ip)
            j //= 2
        k *= 2

    os_ref[...] = sv
    ol_ref[...] = iv % C
    ob_ref[...] = iv // C


def _sort_call(s3, i3):
    return pl.pallas_call(
        _sort_body,
        out_shape=[jax.ShapeDtypeStruct((B, RROW, 128), jnp.float32),
                   jax.ShapeDtypeStruct((B, RROW, 128), jnp.int32),
                   jax.ShapeDtypeStruct((B, RROW, 128), jnp.int32)],
    )(s3, i3)


QPW = Q // SPB
QPW_PAD = QPW + L - (QPW % L)


def _box_body(boxes, bidx, scale, out, table, idxbuf, outbuf, scalev):
    b, w, _, _ = _worker()
    iota = lax.iota(jnp.int32, L)
    pltpu.sync_copy(boxes.at[pl.ds(b * Q * 4, Q * 4)], table)
    idxbuf[pl.ds(QPW_PAD - L, L)] = jnp.zeros((L,), jnp.int32)
    pltpu.sync_copy(bidx.at[pl.ds(b * Q + w * QPW, QPW)],
                    idxbuf.at[pl.ds(0, QPW)])
    pltpu.sync_copy(scale.at[pl.ds(b * 2 * L, 2 * L)], scalev)
    sx = scalev[pl.ds(0, L)]
    sy = scalev[pl.ds(L, L)]

    def vbody(i, _):
        qv = idxbuf[pl.ds(i * L, L)]
        a0 = qv * 4
        cx = plsc.load_gather(table, [a0])
        cy = plsc.load_gather(table, [a0 + 1])
        bw = plsc.load_gather(table, [a0 + 2])
        bh = plsc.load_gather(table, [a0 + 3])
        x0 = (cx - 0.5 * bw) * sx
        y0 = (cy - 0.5 * bh) * sy
        x1 = (cx + 0.5 * bw) * sx
        y1 = (cy + 0.5 * bh) * sy
        p0 = (jnp.broadcast_to(i * L, (L,)) + iota) * 4
        plsc.store_scatter(outbuf, [p0], x0)
        plsc.store_scatter(outbuf, [p0 + 1], y0)
        plsc.store_scatter(outbuf, [p0 + 2], x1)
        plsc.store_scatter(outbuf, [p0 + 3], y1)
        return 0
    lax.fori_loop(0, QPW_PAD // L, vbody, 0)

    pltpu.sync_copy(outbuf.at[pl.ds(0, QPW * 4)],
                    out.at[pl.ds(b * Q * 4 + w * QPW * 4, QPW * 4)])


_box_call = functools.partial(
    pl.kernel,
    out_type=jax.ShapeDtypeStruct((B * Q * 4,), jnp.float32),
    mesh=_MESH,
    compiler_params=_SC_PARAMS,
    scratch_types=[pltpu.VMEM((Q * 4,), jnp.float32),
                   pltpu.VMEM((QPW_PAD,), jnp.int32),
                   pltpu.VMEM((QPW_PAD * 4,), jnp.float32),
                   pltpu.VMEM((2 * L,), jnp.float32)],
)(_box_body)


def kernel(pred_logits, pred_boxes, target_sizes):
    lg = lax.bitcast_convert_type(pred_logits, jnp.int32).reshape(B * N)

    hist = _hist_call(lg)
    hsum = hist.reshape(B, SPB, NBINS).sum(axis=1, dtype=jnp.int32)
    sfx = jnp.cumsum(hsum[:, ::-1], axis=1)[:, ::-1]
    m = jnp.sum((sfx >= Q).astype(jnp.int32), axis=1)
    tbin = jnp.maximum(m - 2, 0)
    th = jnp.left_shift(tbin - 32768, 16).astype(jnp.int32)
    th_splat = jnp.broadcast_to(th[:, None], (B, L)).reshape(B * L)

    cbits, cidx = _compact_call(lg, th_splat)
    cval = lax.bitcast_convert_type(cbits.reshape(B, NCAND), jnp.float32)
    cidx = cidx.reshape(B, NCAND)
    cscore = jax.nn.sigmoid(cval)

    ss, ll, bb = _sort_call(cscore.reshape(B, RROW, 128),
                            cidx.reshape(B, RROW, 128))
    scores = ss.reshape(B, NCAND)[:, :Q]
    labels = ll.reshape(B, NCAND)[:, :Q]
    bidx = bb.reshape(B, NCAND)[:, :Q]

    img_h = target_sizes[:, 0].astype(jnp.float32)
    img_w = target_sizes[:, 1].astype(jnp.float32)
    scale = jnp.stack([img_w, img_h], axis=1)
    scale64 = jnp.broadcast_to(scale[:, :, None], (B, 2, L)).reshape(B * 2 * L)

    boxes = _box_call(pred_boxes.reshape(B * Q * 4), bidx.reshape(B * Q),
                      scale64)
    return scores, labels, boxes.reshape(B, Q, 4)


# speedup vs baseline: 1.0584x; 1.0584x over previous
import functools

import numpy as np
import jax
import jax.numpy as jnp
from jax import lax
from jax.experimental import pallas as pl
from jax.experimental.pallas import tpu as pltpu
from jax.experimental.pallas import tpu_sc as plsc

B = 8
Q = 20000
C = 91
N = Q * C
NC, NS, L = 2, 16, 16
NW = NC * NS
SPB = NW // B
VPB = N // L
VPW = -(-VPB // SPB)
SPAN = VPW * L
CHUNK_V = 512
FULL_CHUNKS, TAIL_V = divmod(VPW, CHUNK_V)
NBINS = 1 << 16
CAP = 8192
NCAND = SPB * CAP
RROW = NCAND // 128
PAD_BITS = int(np.float32(-1e30).view(np.int32))
PAD_IDX = 1 << 30

_MESH = plsc.VectorSubcoreMesh(core_axis_name="c", subcore_axis_name="s",
                               num_cores=NC, num_subcores=NS)
_SC_PARAMS = pltpu.CompilerParams(needs_layout_passes=False)


def _worker():
    c = lax.axis_index("c")
    s = lax.axis_index("s")
    wid = c * NS + s
    b = wid // SPB
    w = wid % SPB
    dma_base = jnp.minimum(w * VPW, VPB - VPW) * L
    own_start = w * SPAN
    return b, w, dma_base, own_start


def _mono_key(bi):
    return bi ^ (jnp.right_shift(bi, 31) & jnp.int32(0x7FFFFFFF))


PRE_V = SPB * VPW - VPB
UNROLL = 1
PREFIX = max(PRE_V, UNROLL)


def _hist_body(lg, hist_out, stage, hist_v):
    b, w, dma_base, own_start = _worker()
    zeros = jnp.zeros((L,), jnp.int32)
    ones = jnp.ones((L,), jnp.int32)

    def zbody(i, _):
        hist_v[pl.ds(i * L, L)] = zeros
        return 0
    lax.fori_loop(0, NBINS // L, zbody, 0)

    def vstep(j, mask=None):
        v = stage[pl.ds(j * L, L)]
        binv = jnp.right_shift(_mono_key(v), 16) + jnp.int32(32768)
        if mask is None:
            plsc.addupdate_scatter(hist_v, [binv], ones)
        else:
            plsc.addupdate_scatter(hist_v, [binv], ones, mask=mask)

    def process(chunk_v, nv, first=False):
        pltpu.sync_copy(lg.at[pl.ds(b * N + dma_base + chunk_v * L, nv * L)],
                        stage.at[pl.ds(0, nv * L)])
        lo = 0
        if first:
            for j in range(PREFIX):
                valid = jnp.broadcast_to(dma_base + j * L >= own_start, (L,))
                vstep(j, mask=valid)
            lo = PREFIX
        n4 = (nv - lo) // UNROLL

        def vbody(i, _):
            j0 = lo + i * UNROLL
            for u in range(UNROLL):
                vstep(j0 + u)
            return 0
        lax.fori_loop(0, n4, vbody, 0)
        for j in range(lo + n4 * UNROLL, nv):
            vstep(j)

    process(0, CHUNK_V, first=True)

    def outer(i, _):
        process(i * CHUNK_V, CHUNK_V)
        return 0
    lax.fori_loop(1, FULL_CHUNKS, outer, 0)
    process(FULL_CHUNKS * CHUNK_V, TAIL_V)

    wid = b * SPB + w
    pltpu.sync_copy(hist_v, hist_out.at[pl.ds(wid * NBINS, NBINS)])


_hist_call = functools.partial(
    pl.kernel,
    out_type=jax.ShapeDtypeStruct((NW * NBINS,), jnp.int32),
    mesh=_MESH,
    compiler_params=_SC_PARAMS,
    scratch_types=[pltpu.VMEM((CHUNK_V * L,), jnp.int32),
                   pltpu.VMEM((NBINS,), jnp.int32)],
)(_hist_body)


def _compact_body(lg, thr, cval_out, cidx_out, stage, cv, ci, thr_v):
    b, w, dma_base, own_start = _worker()
    iota = lax.iota(jnp.int32, L)
    pltpu.sync_copy(thr.at[pl.ds(b * L, L)], thr_v)
    tv = thr_v[...]

    padv = jnp.full((L,), PAD_BITS, jnp.int32)
    padi = jnp.full((L,), PAD_IDX, jnp.int32)

    def pbody(i, _):
        cv[pl.ds(i * L, L)] = padv
        ci[pl.ds(i * L, L)] = padi
        return 0
    lax.fori_loop(0, (CAP + L) // L, pbody, 0)

    def vstep(chunk_v, j, ptr, first=False):
        v = stage[pl.ds(j * L, L)]
        su = _mono_key(v)
        g0 = dma_base + (chunk_v + j) * L
        gidx = jnp.broadcast_to(g0, (L,)) + iota
        keep = su >= tv
        if first:
            keep = keep & jnp.broadcast_to(g0 >= own_start, (L,))
        plsc.store_compressed(cv.at[pl.ds(ptr, L)], v, mask=keep)
        plsc.store_compressed(ci.at[pl.ds(ptr, L)], gidx, mask=keep)
        cnt = jnp.sum(keep.astype(jnp.int32))
        return jnp.minimum(ptr + cnt, CAP)

    def process(chunk_v, nv, ptr, first=False):
        pltpu.sync_copy(lg.at[pl.ds(b * N + dma_base + chunk_v * L, nv * L)],
                        stage.at[pl.ds(0, nv * L)])
        lo = 0
        if first:
            for j in range(PREFIX):
                ptr = vstep(chunk_v, j, ptr, first=True)
            lo = PREFIX
        n4 = (nv - lo) // UNROLL

        def vbody(i, ptr):
            j0 = lo + i * UNROLL
            for u in range(UNROLL):
                ptr = vstep(chunk_v, j0 + u, ptr)
            return ptr
        ptr = lax.fori_loop(0, n4, vbody, ptr)
        for j in range(lo + n4 * UNROLL, nv):
            ptr = vstep(chunk_v, j, ptr)
        return ptr

    ptr = process(0, CHUNK_V, 0, first=True)

    def outer(i, ptr):
        return process(i * CHUNK_V, CHUNK_V, ptr)
    ptr = lax.fori_loop(1, FULL_CHUNKS, outer, ptr)
    process(FULL_CHUNKS * CHUNK_V, TAIL_V, ptr)

    pltpu.sync_copy(cv.at[pl.ds(0, CAP)],
                    cval_out.at[pl.ds(b * NCAND + w * CAP, CAP)])
    pltpu.sync_copy(ci.at[pl.ds(0, CAP)],
                    cidx_out.at[pl.ds(b * NCAND + w * CAP, CAP)])


_compact_call = functools.partial(
    pl.kernel,
    out_type=[jax.ShapeDtypeStruct((B * NCAND,), jnp.int32),
              jax.ShapeDtypeStruct((B * NCAND,), jnp.int32)],
    mesh=_MESH,
    compiler_params=_SC_PARAMS,
    scratch_types=[pltpu.VMEM((CHUNK_V * L,), jnp.int32),
                   pltpu.VMEM((CAP + L,), jnp.int32),
                   pltpu.VMEM((CAP + L,), jnp.int32),
                   pltpu.VMEM((L,), jnp.int32)],
)(_compact_body)


def _sort_body(s_ref, i_ref, os_ref, ol_ref, ob_ref):
    sv = s_ref[...]
    iv = i_ref[...]
    shape = (1, RROW, 128)
    lane = lax.broadcasted_iota(jnp.int32, shape, 2)
    pos = lax.broadcasted_iota(jnp.int32, shape, 1) * 128 + lane

    def xroll(x, j):
        if j >= 128 or j <= -128:
            return jnp.roll(x, j // 128, axis=1)
        return jnp.roll(x, j, axis=2)

    k = 2
    while k <= NCAND:
        j = k // 2
        while j >= 1:
            up = (pos & j) != 0
            desc = (pos & k) != 0
            sp = jnp.where(up, xroll(sv, j), xroll(sv, -j))
            ip = jnp.where(up, xroll(iv, j), xroll(iv, -j))
            prec = (sv > sp) | ((sv == sp) & (iv < ip))
            keep = prec ^ up ^ desc
            sv = jnp.where(keep, sv, sp)
            iv = jnp.where(keep, iv, ip)
            j //= 2
        k *= 2

    os_ref[...] = sv
    ol_ref[...] = iv % C
    ob_ref[...] = iv // C


def _sort_call(s3, i3):
    return pl.pallas_call(
        _sort_body,
        out_shape=[jax.ShapeDtypeStruct((B, RROW, 128), jnp.float32),
                   jax.ShapeDtypeStruct((B, RROW, 128), jnp.int32),
                   jax.ShapeDtypeStruct((B, RROW, 128), jnp.int32)],
    )(s3, i3)


QPW = Q // SPB
QPW_PAD = QPW + L - (QPW % L)


def _box_body(boxes, bidx, scale, out, table, idxbuf, outbuf, scalev):
    b, w, _, _ = _worker()
    iota = lax.iota(jnp.int32, L)
    pltpu.sync_copy(boxes.at[pl.ds(b * Q * 4, Q * 4)], table)
    idxbuf[pl.ds(QPW_PAD - L, L)] = jnp.zeros((L,), jnp.int32)
    pltpu.sync_copy(bidx.at[pl.ds(b * Q + w * QPW, QPW)],
                    idxbuf.at[pl.ds(0, QPW)])
    pltpu.sync_copy(scale.at[pl.ds(b * 2 * L, 2 * L)], scalev)
    sx = scalev[pl.ds(0, L)]
    sy = scalev[pl.ds(L, L)]

    def vbody(i, _):
        qv = idxbuf[pl.ds(i * L, L)]
        a0 = qv * 4
        cx = plsc.load_gather(table, [a0])
        cy = plsc.load_gather(table, [a0 + 1])
        bw = plsc.load_gather(table, [a0 + 2])
        bh = plsc.load_gather(table, [a0 + 3])
        x0 = (cx - 0.5 * bw) * sx
        y0 = (cy - 0.5 * bh) * sy
        x1 = (cx + 0.5 * bw) * sx
        y1 = (cy + 0.5 * bh) * sy
        p0 = (jnp.broadcast_to(i * L, (L,)) + iota) * 4
        plsc.store_scatter(outbuf, [p0], x0)
        plsc.store_scatter(outbuf, [p0 + 1], y0)
        plsc.store_scatter(outbuf, [p0 + 2], x1)
        plsc.store_scatter(outbuf, [p0 + 3], y1)
        return 0
    lax.fori_loop(0, QPW_PAD // L, vbody, 0)

    pltpu.sync_copy(outbuf.at[pl.ds(0, QPW * 4)],
                    out.at[pl.ds(b * Q * 4 + w * QPW * 4, QPW * 4)])


_box_call = functools.partial(
    pl.kernel,
    out_type=jax.ShapeDtypeStruct((B * Q * 4,), jnp.float32),
    mesh=_MESH,
    compiler_params=_SC_PARAMS,
    scratch_types=[pltpu.VMEM((Q * 4,), jnp.float32),
                   pltpu.VMEM((QPW_PAD,), jnp.int32),
                   pltpu.VMEM((QPW_PAD * 4,), jnp.float32),
                   pltpu.VMEM((2 * L,), jnp.float32)],
)(_box_body)


def kernel(pred_logits, pred_boxes, target_sizes):
    lg = lax.bitcast_convert_type(pred_logits, jnp.int32).reshape(B * N)

    hist = _hist_call(lg)
    hsum = hist.reshape(B, SPB, NBINS).sum(axis=1, dtype=jnp.int32)
    sfx = jnp.cumsum(hsum[:, ::-1], axis=1)[:, ::-1]
    m = jnp.sum((sfx >= Q).astype(jnp.int32), axis=1)
    tbin = jnp.maximum(m - 2, 0)
    th = jnp.left_shift(tbin - 32768, 16).astype(jnp.int32)
    th_splat = jnp.broadcast_to(th[:, None], (B, L)).reshape(B * L)

    cbits, cidx = _compact_call(lg, th_splat)
    cval = lax.bitcast_convert_type(cbits.reshape(B, NCAND), jnp.float32)
    cidx = cidx.reshape(B, NCAND)
    cscore = jax.nn.sigmoid(cval)

    ss, ll, bb = _sort_call(cscore.reshape(B, RROW, 128),
                            cidx.reshape(B, RROW, 128))
    scores = ss.reshape(B, NCAND)[:, :Q]
    labels = ll.reshape(B, NCAND)[:, :Q]
    bidx = bb.reshape(B, NCAND)[:, :Q]

    img_h = target_sizes[:, 0].astype(jnp.float32)
    img_w = target_sizes[:, 1].astype(jnp.float32)
    scale = jnp.stack([img_w, img_h], axis=1)
    scale64 = jnp.broadcast_to(scale[:, :, None], (B, 2, L)).reshape(B * 2 * L)

    boxes = _box_call(pred_boxes.reshape(B * Q * 4), bidx.reshape(B * Q),
                      scale64)
    return scores, labels, boxes.reshape(B, Q, 4)
